# Initial kernel scaffold; baseline (speedup 1.0000x reference)
import jax
import jax.numpy as jnp
from jax.experimental import pallas as pl


def kernel(x, edge_index, batch, W_init, b_init, W0a, b0a, W0b, b0b, W1a, b1a, W1b, b1b, W2a, b2a, W2b, b2b, Wp1, bp1, Wp2, bp2):
    raise NotImplementedError("write your pallas kernel here")


# speedup vs baseline: 4.2717x; 4.2717x over previous
import functools

import jax
import jax.numpy as jnp
from jax import lax
from jax.experimental import pallas as pl
from jax.experimental.pallas import tpu as pltpu
from jax.experimental.pallas import tpu_sc as plsc

N = 10000
E = 320000
F_IN = 30
H = 128
NUM_GRAPHS = 64

NC = 2
NS = 16
NW = NC * NS
EPT = E // NW
CHUNK = 80
NCHUNK = EPT // CHUNK
ROWS_PT = N // NS
ZR = 25

ROW_BLK = 1000
GRID = N // ROW_BLK

_sc_mesh = plsc.VectorSubcoreMesh(core_axis_name="c", subcore_axis_name="s")


@functools.partial(
    pl.kernel,
    mesh=_sc_mesh,
    out_type=(
        jax.ShapeDtypeStruct((N, H), jnp.float32),
        jax.ShapeDtypeStruct((N, H), jnp.float32),
    ),
    scratch_types=[
        pltpu.VMEM_SHARED((N, H), jnp.float32),
        pltpu.VMEM((CHUNK,), jnp.int32),
        pltpu.VMEM((CHUNK,), jnp.int32),
        pltpu.VMEM((CHUNK, H), jnp.float32),
        pltpu.VMEM((ZR, H), jnp.float32),
        pltpu.SemaphoreType.DMA,
    ],
)
def _sc_edge_aggr(h_hbm, src_hbm, dst_hbm, out0_hbm, out1_hbm,
                  acc, sidx, didx, rows, zbuf, sem):
    cid = lax.axis_index("c")
    sid = lax.axis_index("s")
    wid = sid * NC + cid

    zeros16 = jnp.zeros((16,), jnp.float32)

    def _zero_row(r, carry):
        for j in range(H // 16):
            zbuf[r, pl.ds(j * 16, 16)] = zeros16
        return carry

    lax.fori_loop(0, ZR, _zero_row, 0)

    def _zero_acc(k, carry):
        pltpu.sync_copy(zbuf, acc.at[pl.ds(sid * ROWS_PT + k * ZR, ZR)])
        return carry

    lax.fori_loop(0, ROWS_PT // ZR, _zero_acc, 0)
    plsc.subcore_barrier()

    base_e = wid * EPT

    def _chunk(i, carry):
        off = base_e + i * CHUNK
        pltpu.sync_copy(src_hbm.at[pl.ds(off, CHUNK)], sidx)
        pltpu.sync_copy(dst_hbm.at[pl.ds(off, CHUNK)], didx)
        pltpu.async_copy(h_hbm.at[sidx], rows, sem).wait()
        pltpu.sync_copy(rows, acc.at[didx], add=True)
        return carry

    lax.fori_loop(0, NCHUNK, _chunk, 0)
    plsc.subcore_barrier()

    my_rows = pl.ds(sid * 624, 624)
    tail = pl.ds(N - 16, 16)

    @pl.when(cid == 0)
    def _():
        pltpu.sync_copy(acc.at[my_rows], out0_hbm.at[my_rows])

        @pl.when(sid == NS - 1)
        def _():
            pltpu.sync_copy(acc.at[tail], out0_hbm.at[tail])

    @pl.when(cid == 1)
    def _():
        pltpu.sync_copy(acc.at[my_rows], out1_hbm.at[my_rows])

        @pl.when(sid == NS - 1)
        def _():
            pltpu.sync_copy(acc.at[tail], out1_hbm.at[tail])


def _init_body(x_ref, w_ref, b_ref, o_ref):
    o_ref[...] = (
        jnp.dot(x_ref[...], w_ref[...], preferred_element_type=jnp.float32)
        + b_ref[...]
    )


def _mlp_body(h_ref, a0_ref, a1_ref, wa_ref, ba_ref, wb_ref, bb_ref, o_ref,
              *, last):
    z = h_ref[...] + a0_ref[...] + a1_ref[...]
    z = jnp.dot(z, wa_ref[...], preferred_element_type=jnp.float32) + ba_ref[...]
    z = jnp.maximum(z, 0.0)
    z = jnp.dot(z, wb_ref[...], preferred_element_type=jnp.float32) + bb_ref[...]
    if not last:
        z = jnp.maximum(z, 0.0)
    o_ref[...] = z


def _readout_body(h_ref, b_ref, wp1_ref, bp1_ref, wp2_ref, bp2_ref, o_ref,
                  g_acc):
    i = pl.program_id(0)

    @pl.when(i == 0)
    def _():
        g_acc[...] = jnp.zeros_like(g_acc)

    bids = b_ref[0, 0, :]
    onehot = (
        bids[:, None] == lax.broadcasted_iota(jnp.int32, (ROW_BLK, NUM_GRAPHS), 1)
    ).astype(jnp.float32)
    g_acc[...] += lax.dot_general(
        onehot, h_ref[...], (((0,), (0,)), ((), ())),
        preferred_element_type=jnp.float32,
    )

    @pl.when(i == pl.num_programs(0) - 1)
    def _():
        g = g_acc[...]
        y = jnp.maximum(
            jnp.dot(g, wp1_ref[...], preferred_element_type=jnp.float32)
            + bp1_ref[...],
            0.0,
        )
        o_ref[...] = (
            jnp.dot(y, wp2_ref[...], preferred_element_type=jnp.float32)
            + bp2_ref[...]
        )


_full = lambda shape: pl.BlockSpec(shape, lambda i: (0,) * len(shape))
_rows = lambda w: pl.BlockSpec((ROW_BLK, w), lambda i: (i, 0))

_init_call = pl.pallas_call(
    _init_body,
    grid=(GRID,),
    in_specs=[_rows(32), _full((32, H)), _full((1, H))],
    out_specs=_rows(H),
    out_shape=jax.ShapeDtypeStruct((N, H), jnp.float32),
)


def _mlp_call(last):
    return pl.pallas_call(
        functools.partial(_mlp_body, last=last),
        grid=(GRID,),
        in_specs=[_rows(H), _rows(H), _rows(H),
                  _full((H, H)), _full((1, H)), _full((H, H)), _full((1, H))],
        out_specs=_rows(H),
        out_shape=jax.ShapeDtypeStruct((N, H), jnp.float32),
    )


_readout_call = pl.pallas_call(
    _readout_body,
    grid=(GRID,),
    in_specs=[_rows(H), pl.BlockSpec((1, 1, ROW_BLK), lambda i: (i, 0, 0)),
              _full((H, H)), _full((1, H)), _full((H, H)), _full((1, H))],
    out_specs=_full((NUM_GRAPHS, H)),
    out_shape=jax.ShapeDtypeStruct((NUM_GRAPHS, H), jnp.float32),
    scratch_shapes=[pltpu.VMEM((NUM_GRAPHS, H), jnp.float32)],
)


def kernel(x, edge_index, batch, W_init, b_init, W0a, b0a, W0b, b0b,
           W1a, b1a, W1b, b1b, W2a, b2a, W2b, b2b, Wp1, bp1, Wp2, bp2):
    src = edge_index[0]
    dst = edge_index[1]
    x_pad = jnp.pad(x, ((0, 0), (0, 32 - F_IN)))
    w_pad = jnp.pad(W_init, ((0, 32 - F_IN), (0, 0)))

    h = _init_call(x_pad, w_pad, b_init.reshape(1, H))

    layers = [
        (W0a, b0a, W0b, b0b, False),
        (W1a, b1a, W1b, b1b, False),
        (W2a, b2a, W2b, b2b, True),
    ]
    for Wa, ba, Wb, bb, last in layers:
        a0, a1 = _sc_edge_aggr(h, src, dst)
        h = _mlp_call(last)(h, a0, a1, Wa, ba.reshape(1, H),
                            Wb, bb.reshape(1, H))

    batch3 = batch.reshape(GRID, 1, ROW_BLK)
    wp2_pad = jnp.pad(Wp2, ((0, 0), (0, H - 1)))
    bp2_pad = jnp.pad(bp2, ((0, H - 1)))
    out = _readout_call(h, batch3, Wp1, bp1.reshape(1, H),
                        wp2_pad, bp2_pad.reshape(1, H))
    return out[:, :1]
